# baseline (speedup 1.0000x reference)
import functools

import jax
import jax.numpy as jnp
from jax import lax
from jax.experimental import pallas as pl
from jax.experimental.pallas import tpu as pltpu
from jax.experimental.pallas import tpu_sc as plsc

FE, FX, FU, FOUT = 16, 128, 16, 2
N, E = 10000, 320000
L = 16
NC, NS = 2, 16
NW = NC * NS
EW = E // NW
C = 400
NCHUNK = EW // C
NP = 10240
NPT = NP // NS

f32 = jnp.float32


def _make_edge_sc(with_count):
    mesh = plsc.VectorSubcoreMesh(core_axis_name="c", subcore_axis_name="s")

    out_type = [jax.ShapeDtypeStruct((NC * NP, L), f32)]
    if with_count:
        out_type.append(jax.ShapeDtypeStruct((E // 8, 8 * L), f32))
        out_type.append(jax.ShapeDtypeStruct((NC * NP, L), f32))

    scratch = (
        [pltpu.VMEM((C,), jnp.int32) for _ in range(2)]
        + [pltpu.VMEM((C,), jnp.int32) for _ in range(3)]
        + [pltpu.VMEM((C // 8, 8 * L), f32) for _ in range(2)]
        + [pltpu.VMEM((C, L), f32) for _ in range(2)]
        + [pltpu.VMEM((C, L), f32) for _ in range(2)]
        + [pltpu.VMEM((C, L), f32) for _ in range(3)]
        + [pltpu.VMEM((C // 8, 8 * L), f32) for _ in range(3)]
        + [
            pltpu.VMEM((L,), f32),
            pltpu.VMEM((NPT, L), f32),
            pltpu.VMEM_SHARED((NP, L), f32),
        ]
    )
    n_sem = 2 + 3 + 2 + 2 + 2 + 3
    if with_count:
        scratch.append(pltpu.VMEM((C, L), f32))
        scratch.append(pltpu.VMEM_SHARED((NP, L), f32))
        n_sem += 3 + 3
    scratch += [pltpu.SemaphoreType.DMA for _ in range(n_sem)]

    @functools.partial(
        pl.kernel, mesh=mesh, out_type=out_type, scratch_types=scratch,
        compiler_params=pltpu.CompilerParams(use_tc_tiling_on_sc=False))
    def edge_sc(src_hbm, dst_hbm, ep_hbm, xs_hbm, xd_hbm, ut_hbm, *rest):
        if with_count:
            ssum_hbm, r_hbm, cnt_hbm = rest[:3]
            rest = rest[3:]
        else:
            ssum_hbm = rest[0]
            rest = rest[1:]
        src_v = rest[0:2]
        dst_v = rest[2:5]
        ep_v = rest[5:7]
        xs_v = rest[7:9]
        xd_v = rest[9:11]
        r_v = rest[11:14]
        rb_v = rest[14:17]
        ut_v, z_v, acc = rest[17:20]
        rest = rest[20:]
        if with_count:
            ones_v, cacc = rest[:2]
            rest = rest[2:]
        sem_src = rest[0:2]
        sem_dst = rest[2:5]
        sem_ep = rest[5:7]
        sem_gx = rest[7:9]
        sem_gd = rest[9:11]
        sem_sc = rest[11:14]
        if with_count:
            sem_cnt = rest[14:17]
            sem_r = rest[17:20]

        cid = lax.axis_index("c")
        sid = lax.axis_index("s")
        wid = sid * NC + cid

        def zf(i, carry):
            z_v[i] = jnp.zeros((L,), f32)
            return carry
        lax.fori_loop(0, NPT, zf, 0)
        pltpu.sync_copy(z_v, acc.at[pl.ds(sid * NPT, NPT)])
        if with_count:
            def of(i, carry):
                ones_v[i] = jnp.full((L,), 1.0, f32)
                return carry
            lax.fori_loop(0, C, of, 0)
            pltpu.sync_copy(z_v, cacc.at[pl.ds(sid * NPT, NPT)])
        pltpu.sync_copy(ut_hbm, ut_v)
        ut = ut_v[...]
        plsc.subcore_barrier()

        d_src, d_dst, d_ep, d_gx, d_gd = {}, {}, {}, {}, {}
        d_sc, d_cnt, d_r = {}, {}, {}

        def bb(k):
            return pl.multiple_of(wid * EW + k * C, C)

        def issue_loads(j):
            if j - 3 in d_sc:
                d_sc.pop(j - 3).wait()
            if j - 3 in d_cnt:
                d_cnt.pop(j - 3).wait()
            if j - 3 in d_r:
                d_r.pop(j - 3).wait()
            base = bb(j)
            s2, s3 = j % 2, j % 3
            d_src[j] = pltpu.async_copy(
                src_hbm.at[pl.ds(base, C)], src_v[s2], sem_src[s2])
            d_dst[j] = pltpu.async_copy(
                dst_hbm.at[pl.ds(base, C)], dst_v[s3], sem_dst[s3])
            d_ep[j] = pltpu.async_copy(
                ep_hbm.at[pl.ds(base // 8, C // 8)], ep_v[s2], sem_ep[s2])

        def issue_gathers(j):
            s2, s3 = j % 2, j % 3
            d_src.pop(j).wait()
            d_dst.pop(j).wait()
            d_gx[j] = pltpu.async_copy(
                xs_hbm.at[src_v[s2]], xs_v[s2], sem_gx[s2])
            d_gd[j] = pltpu.async_copy(
                xd_hbm.at[dst_v[s3]], xd_v[s2], sem_gd[s2])

        issue_loads(0)
        issue_gathers(0)
        for k in range(NCHUNK):
            s2, s3 = k % 2, k % 3
            if k + 1 < NCHUNK:
                issue_loads(k + 1)
            d_gx.pop(k).wait()
            d_gd.pop(k).wait()
            d_ep.pop(k).wait()
            if k + 1 < NCHUNK:
                issue_gathers(k + 1)

            xs_s, xd_s, ep_s, r_s = xs_v[s2], xd_v[s2], ep_v[s2], r_v[s3]
            rb_s = rb_v[s3]

            if with_count:
                @plsc.parallel_loop(0, C // 8, unroll=1)
                def body(row):
                    for j in range(8):
                        i = row * 8 + j
                        v = jnp.maximum(
                            xs_s[i] + xd_s[i]
                            + ep_s[row, pl.ds(L * j, L)] + ut, 0.0)
                        r_s[i] = v
                        rb_s[row, pl.ds(L * j, L)] = v
            else:
                @plsc.parallel_loop(0, C // 8, unroll=1)
                def body(row):
                    for j in range(8):
                        i = row * 8 + j
                        r_s[i] = jnp.maximum(
                            xs_s[i] + xd_s[i]
                            + ep_s[row, pl.ds(L * j, L)] + ut, 0.0)

            d_sc[k] = pltpu.async_copy(
                r_s, acc.at[dst_v[s3]], sem_sc[s3], add=True)
            if with_count:
                d_cnt[k] = pltpu.async_copy(
                    ones_v, cacc.at[dst_v[s3]], sem_cnt[s3], add=True)
                d_r[k] = pltpu.async_copy(
                    rb_s, r_hbm.at[pl.ds(bb(k) // 8, C // 8)], sem_r[s3])

        for j in sorted(d_sc):
            d_sc.pop(j).wait()
        for j in sorted(d_cnt):
            d_cnt.pop(j).wait()
        for j in sorted(d_r):
            d_r.pop(j).wait()

        plsc.subcore_barrier()
        pltpu.sync_copy(acc.at[pl.ds(sid * NPT, NPT)], z_v)
        pltpu.sync_copy(z_v, ssum_hbm.at[pl.ds(cid * NP + sid * NPT, NPT)])
        if with_count:
            pltpu.sync_copy(cacc.at[pl.ds(sid * NPT, NPT)], z_v)
            pltpu.sync_copy(z_v, cnt_hbm.at[pl.ds(cid * NP + sid * NPT, NPT)])

    return edge_sc


_edge_sc_full = _make_edge_sc(True)
_edge_sc_slim = _make_edge_sc(False)


_IDC = 250


def _make_ident_sc():
    mesh = plsc.VectorSubcoreMesh(core_axis_name="c", subcore_axis_name="s")
    rw = (E // 8) // NW
    out_type = [jax.ShapeDtypeStruct((E // 8, 8 * L), f32),
                jax.ShapeDtypeStruct((E // 8, 8 * L), f32)]
    scratch = ([pltpu.VMEM((_IDC, 8 * L), f32) for _ in range(2)]
               + [pltpu.SemaphoreType.DMA for _ in range(2)])

    @functools.partial(
        pl.kernel, mesh=mesh, out_type=out_type, scratch_types=scratch,
        compiler_params=pltpu.CompilerParams(use_tc_tiling_on_sc=False))
    def ident_sc(ea_hbm, eb_hbm, oa_hbm, ob_hbm, buf0, buf1, sem0, sem1):
        cid = lax.axis_index("c")
        sid = lax.axis_index("s")
        wid = sid * NC + cid
        for src_hbm, dst_hbm in ((ea_hbm, oa_hbm), (eb_hbm, ob_hbm)):
            for k in range(rw // _IDC):
                base = wid * rw + k * _IDC
                sl = pl.ds(base, _IDC)
                buf = (buf0, buf1)[k % 2]
                sem = (sem0, sem1)[k % 2]
                pltpu.async_copy(src_hbm.at[sl], buf, sem).wait()
                pltpu.async_copy(buf, dst_hbm.at[sl], sem).wait()

    return ident_sc


_ident_sc = _make_ident_sc()


def _proj_body(x_ref, wsd_ref, ucat_ref, w0u_ref, eb0_ref,
               xs_ref, xd_ref, ut_ref):
    pj = jnp.dot(x_ref[...], wsd_ref[...], preferred_element_type=f32)
    xs_ref[...] = pj[:, :L]
    xd_ref[...] = pj[:, L:]
    ut_ref[...] = (jnp.dot(ucat_ref[...], w0u_ref[...],
                           preferred_element_type=f32) + eb0_ref[...])


def _proj(x, wsd, ucat, w0u, eb0):
    return pl.pallas_call(
        _proj_body,
        out_shape=[jax.ShapeDtypeStruct((N, L), f32),
                   jax.ShapeDtypeStruct((N, L), f32),
                   jax.ShapeDtypeStruct((1, L), f32)],
    )(x, wsd, ucat, w0u, eb0)


_EP_CHUNK = 4000


def _ep_body(rows_ref, k_ref, pb_ref, o_ref):
    o_ref[...] = (jnp.dot(rows_ref[...], k_ref[...],
                          preferred_element_type=f32) + pb_ref[...])


_EPI_ROWS = 1000


def _ep_init_body(e_ref, w_ref, o_ref):
    for j in range(8):
        o_ref[:, L * j:L * (j + 1)] = jnp.dot(
            e_ref[:, j, :], w_ref[...], preferred_element_type=f32)


def _ep_init(e3, w0e):
    g = e3.shape[0] // _EPI_ROWS
    return pl.pallas_call(
        _ep_init_body,
        grid=(g,),
        in_specs=[pl.BlockSpec((_EPI_ROWS, 8, L), lambda i: (i, 0, 0)),
                  pl.BlockSpec((L, L), lambda i: (0, 0))],
        out_specs=pl.BlockSpec((_EPI_ROWS, 8 * L), lambda i: (i, 0)),
        out_shape=jax.ShapeDtypeStruct((e3.shape[0], 8 * L), f32),
    )(e3, w0e)


def _ep(rows_b, kmat, pb_tile):
    g = rows_b.shape[0] // _EP_CHUNK
    return pl.pallas_call(
        _ep_body,
        grid=(g,),
        in_specs=[pl.BlockSpec((_EP_CHUNK, 8 * L), lambda i: (i, 0)),
                  pl.BlockSpec((8 * L, 8 * L), lambda i: (0, 0)),
                  pl.BlockSpec((1, 8 * L), lambda i: (0, 0))],
        out_specs=pl.BlockSpec((_EP_CHUNK, 8 * L), lambda i: (i, 0)),
        out_shape=jax.ShapeDtypeStruct((rows_b.shape[0], 8 * L), f32),
    )(rows_b, kmat, pb_tile)


def _node_body(ssum_ref, cnt_ref, x_ref, ucat_ref, uoth_ref,
               eW1_ref, eb1_ref, Wx_ref, Wa_ref, Wu_ref, nb0_ref,
               nW1_ref, nb1_ref, Wgu_ref, Wgn_ref, gb0_ref, gW1_ref, gb1_ref,
               W0u_ref, eb0_ref, wsd_ref, oW0_ref, ob0_ref, oW1_ref, ob1_ref,
               xnew_ref, unew_ref, utnext_ref, xs_ref, xd_ref, out_ref):
    s = ssum_ref[:N, :] + ssum_ref[NP:NP + N, :]
    c = cnt_ref[:N, :1] + cnt_ref[NP:NP + N, :1]
    agg = ((jnp.dot(s, eW1_ref[...], preferred_element_type=f32)
            + c * eb1_ref[...]) / jnp.maximum(c, 1.0))
    ucat = ucat_ref[...]
    ut_n = jnp.dot(ucat, Wu_ref[...], preferred_element_type=f32) + nb0_ref[...]
    xh = jnp.maximum(
        jnp.dot(x_ref[...], Wx_ref[...], preferred_element_type=f32)
        + jnp.dot(agg, Wa_ref[...], preferred_element_type=f32) + ut_n, 0.0)
    x_new = jnp.dot(xh, nW1_ref[...], preferred_element_type=f32) + nb1_ref[...]
    xnew_ref[...] = x_new
    na = jnp.sum(x_new, axis=0, keepdims=True) * (1.0 / N)
    hg = jnp.maximum(
        jnp.dot(ucat, Wgu_ref[...], preferred_element_type=f32)
        + jnp.dot(na, Wgn_ref[...], preferred_element_type=f32)
        + gb0_ref[...], 0.0)
    u_new = jnp.dot(hg, gW1_ref[...], preferred_element_type=f32) + gb1_ref[...]
    unew_ref[...] = u_new
    uoth = uoth_ref[...]
    utnext_ref[...] = (
        jnp.dot(uoth, W0u_ref[:L, :], preferred_element_type=f32)
        + jnp.dot(u_new, W0u_ref[L:, :], preferred_element_type=f32)
        + eb0_ref[...])
    pj = jnp.dot(x_new, wsd_ref[...], preferred_element_type=f32)
    xs_ref[...] = pj[:, :L]
    xd_ref[...] = pj[:, L:]
    ho = jnp.maximum(
        jnp.dot(uoth, oW0_ref[:L, :], preferred_element_type=f32)
        + jnp.dot(u_new, oW0_ref[L:, :], preferred_element_type=f32)
        + ob0_ref[...], 0.0)
    out_ref[...] = (jnp.dot(ho, oW1_ref[...], preferred_element_type=f32)
                    + ob1_ref[...])


def _node(ssum, cnt, x, ucat, uoth, w):
    return pl.pallas_call(
        _node_body,
        out_shape=[jax.ShapeDtypeStruct((N, FX), f32),
                   jax.ShapeDtypeStruct((1, L), f32),
                   jax.ShapeDtypeStruct((1, L), f32),
                   jax.ShapeDtypeStruct((N, L), f32),
                   jax.ShapeDtypeStruct((N, L), f32),
                   jax.ShapeDtypeStruct((1, FOUT), f32)],
    )(ssum, cnt, x, ucat, uoth, *w)


def _node_last_body(ssum_ref, cnt_ref, x_ref, ucat_ref, uoth_ref,
                    eW1_ref, eb1_ref, Wx_ref, Wa_ref, Wu_ref, nb0_ref,
                    nW1_ref, nb1_ref, Wgu_ref, Wgn_ref, gb0_ref, gW1_ref,
                    gb1_ref, W0u_ref, eb0_ref, wsd_ref, oW0_ref, ob0_ref,
                    oW1_ref, ob1_ref, unew_ref, utnext_ref, out_ref):
    s = ssum_ref[:N, :] + ssum_ref[NP:NP + N, :]
    c = cnt_ref[:N, :1] + cnt_ref[NP:NP + N, :1]
    agg = ((jnp.dot(s, eW1_ref[...], preferred_element_type=f32)
            + c * eb1_ref[...]) / jnp.maximum(c, 1.0))
    ucat = ucat_ref[...]
    ut_n = jnp.dot(ucat, Wu_ref[...], preferred_element_type=f32) + nb0_ref[...]
    xh = jnp.maximum(
        jnp.dot(x_ref[...], Wx_ref[...], preferred_element_type=f32)
        + jnp.dot(agg, Wa_ref[...], preferred_element_type=f32) + ut_n, 0.0)
    x_new = jnp.dot(xh, nW1_ref[...], preferred_element_type=f32) + nb1_ref[...]
    na = jnp.sum(x_new, axis=0, keepdims=True) * (1.0 / N)
    hg = jnp.maximum(
        jnp.dot(ucat, Wgu_ref[...], preferred_element_type=f32)
        + jnp.dot(na, Wgn_ref[...], preferred_element_type=f32)
        + gb0_ref[...], 0.0)
    u_new = jnp.dot(hg, gW1_ref[...], preferred_element_type=f32) + gb1_ref[...]
    unew_ref[...] = u_new
    uoth = uoth_ref[...]
    utnext_ref[...] = (
        jnp.dot(uoth, W0u_ref[:L, :], preferred_element_type=f32)
        + jnp.dot(u_new, W0u_ref[L:, :], preferred_element_type=f32)
        + eb0_ref[...])
    ho = jnp.maximum(
        jnp.dot(uoth, oW0_ref[:L, :], preferred_element_type=f32)
        + jnp.dot(u_new, oW0_ref[L:, :], preferred_element_type=f32)
        + ob0_ref[...], 0.0)
    out_ref[...] = (jnp.dot(ho, oW1_ref[...], preferred_element_type=f32)
                    + ob1_ref[...])


def _node_last(ssum, cnt, x, ucat, uoth, w):
    return pl.pallas_call(
        _node_last_body,
        out_shape=[jax.ShapeDtypeStruct((1, L), f32),
                   jax.ShapeDtypeStruct((1, L), f32),
                   jax.ShapeDtypeStruct((1, FOUT), f32)],
    )(ssum, cnt, x, ucat, uoth, *w)


def kernel(x1, edge_index1, e1, u1, batch1, x2, edge_index2, e2, u2, batch2,
           edge_W0, edge_b0, edge_W1, edge_b1,
           node_W0, node_b0, node_W1, node_b1,
           glob_W0, glob_b0, glob_W1, glob_b1,
           out_W0, out_b0, out_W1, out_b1):
    src1, dst1 = edge_index1[0], edge_index1[1]
    src2, dst2 = edge_index2[0], edge_index2[1]

    wsd = jnp.concatenate([edge_W0[:FX], edge_W0[FX:2 * FX]], axis=1)
    w0e = edge_W0[2 * FX:2 * FX + FE]
    w0u = edge_W0[2 * FX + FE:]
    eb0 = edge_b0.reshape(1, L)
    eb1 = edge_b1.reshape(1, L)
    eye8 = jnp.eye(8, dtype=f32)
    k_init = jnp.kron(eye8, w0e)
    k_step = jnp.kron(eye8, edge_W1 @ w0e)
    pb_init = jnp.zeros((1, 8 * L), f32)
    pb_step = jnp.tile((edge_b1 @ w0e).reshape(1, L), (1, 8))
    wx = node_W0[:FX]
    wa = node_W0[FX:FX + FE]
    wu = node_W0[FX + FE:]
    nb0 = node_b0.reshape(1, L)
    nb1 = node_b1.reshape(1, FX)
    wgu = glob_W0[:2 * FU]
    wgn = glob_W0[2 * FU:]
    gb0 = glob_b0.reshape(1, L)
    gb1 = glob_b1.reshape(1, L)
    ob0 = out_b0.reshape(1, L)
    ob1 = out_b1.reshape(1, FOUT)
    nodew = (edge_W1, eb1, wx, wa, wu, nb0, node_W1, nb1,
             wgu, wgn, gb0, glob_W1, gb1, w0u, eb0, wsd,
             out_W0, ob0, out_W1, ob1)

    ucat11 = jnp.concatenate([u1, u2], axis=1)
    xs1, xd1, ut11 = _proj(x1, wsd, ucat11, w0u, eb0)
    xs2, xd2, _ = _proj(x2, wsd, ucat11, w0u, eb0)
    e1b, e2b = _ident_sc(e1.reshape(E // 8, 8 * L), e2.reshape(E // 8, 8 * L))
    ep1 = _ep(e1b, k_init, pb_init)
    ep2 = _ep(e2b, k_init, pb_init)

    ssum1, r1, cnt1 = _edge_sc_full(src1, dst1, ep1, xs1, xd1,
                                    ut11.reshape(L))
    x1b, u1b, ut21, xs1b, xd1b, _ = _node(ssum1, cnt1, x1, ucat11, u2, nodew)

    ssum2, r2, cnt2 = _edge_sc_full(src2, dst2, ep2, xs2, xd2,
                                    ut21.reshape(L))
    ucat21 = jnp.concatenate([u2, u1b], axis=1)
    x2b, u2b, ut12, xs2b, xd2b, out1 = _node(ssum2, cnt2, x2, ucat21, u1b,
                                             nodew)

    ep1b = _ep(r1, k_step, pb_step)
    (ssum1b,) = _edge_sc_slim(src1, dst1, ep1b, xs1b, xd1b, ut12.reshape(L))
    ucat12 = jnp.concatenate([u1b, u2b], axis=1)
    u1c, ut22, _ = _node_last(ssum1b, cnt1, x1b, ucat12, u2b, nodew)

    ep2b = _ep(r2, k_step, pb_step)
    (ssum2b,) = _edge_sc_slim(src2, dst2, ep2b, xs2b, xd2b, ut22.reshape(L))
    ucat22 = jnp.concatenate([u2b, u1c], axis=1)
    _, _, out2 = _node_last(ssum2b, cnt2, x2b, ucat22, u1c, nodew)

    return jnp.stack([out1, out2])


# speedup vs baseline: 1.1210x; 1.1210x over previous
import functools

import jax
import jax.numpy as jnp
from jax import lax
from jax.experimental import pallas as pl
from jax.experimental.pallas import tpu as pltpu
from jax.experimental.pallas import tpu_sc as plsc

FE, FX, FU, FOUT = 16, 128, 16, 2
N, E = 10000, 320000
L = 16
NC, NS = 2, 16
NW = NC * NS
EW = E // NW
C = 400
NCHUNK = EW // C
NP = 10240
NPT = NP // NS

f32 = jnp.float32


def _make_edge_sc(with_count):
    mesh = plsc.VectorSubcoreMesh(core_axis_name="c", subcore_axis_name="s")

    out_type = [jax.ShapeDtypeStruct((NC * NP, L), f32)]
    if with_count:
        out_type.append(jax.ShapeDtypeStruct((E // 8, 8 * L), f32))
        out_type.append(jax.ShapeDtypeStruct((NC * NP, L), f32))

    scratch = (
        [pltpu.VMEM((C,), jnp.int32) for _ in range(2)]
        + [pltpu.VMEM((C,), jnp.int32) for _ in range(3)]
        + [pltpu.VMEM((C // 8, 8 * L), f32) for _ in range(2)]
        + [pltpu.VMEM((C, L), f32) for _ in range(2)]
        + [pltpu.VMEM((C, L), f32) for _ in range(2)]
        + [pltpu.VMEM((C, L), f32) for _ in range(3)]
        + [pltpu.VMEM((C // 8, 8 * L), f32) for _ in range(3)]
        + [
            pltpu.VMEM((L,), f32),
            pltpu.VMEM((NPT, L), f32),
            pltpu.VMEM_SHARED((NP, L), f32),
        ]
    )
    n_sem = 2 + 3 + 2 + 2 + 2 + 3
    if with_count:
        scratch.append(pltpu.VMEM((C, L), f32))
        scratch.append(pltpu.VMEM_SHARED((NP, L), f32))
        n_sem += 3 + 3
    scratch += [pltpu.SemaphoreType.DMA for _ in range(n_sem)]

    @functools.partial(
        pl.kernel, mesh=mesh, out_type=out_type, scratch_types=scratch,
        compiler_params=pltpu.CompilerParams(use_tc_tiling_on_sc=False))
    def edge_sc(src_hbm, dst_hbm, ep_hbm, xs_hbm, xd_hbm, ut_hbm, *rest):
        if with_count:
            ssum_hbm, r_hbm, cnt_hbm = rest[:3]
            rest = rest[3:]
        else:
            ssum_hbm = rest[0]
            rest = rest[1:]
        src_v = rest[0:2]
        dst_v = rest[2:5]
        ep_v = rest[5:7]
        xs_v = rest[7:9]
        xd_v = rest[9:11]
        r_v = rest[11:14]
        rb_v = rest[14:17]
        ut_v, z_v, acc = rest[17:20]
        rest = rest[20:]
        if with_count:
            ones_v, cacc = rest[:2]
            rest = rest[2:]
        sem_src = rest[0:2]
        sem_dst = rest[2:5]
        sem_ep = rest[5:7]
        sem_gx = rest[7:9]
        sem_gd = rest[9:11]
        sem_sc = rest[11:14]
        if with_count:
            sem_cnt = rest[14:17]
            sem_r = rest[17:20]

        cid = lax.axis_index("c")
        sid = lax.axis_index("s")
        wid = sid * NC + cid

        def zf(i, carry):
            z_v[i] = jnp.zeros((L,), f32)
            return carry
        lax.fori_loop(0, NPT, zf, 0)
        pltpu.sync_copy(z_v, acc.at[pl.ds(sid * NPT, NPT)])
        if with_count:
            def of(i, carry):
                ones_v[i] = jnp.full((L,), 1.0, f32)
                return carry
            lax.fori_loop(0, C, of, 0)
            pltpu.sync_copy(z_v, cacc.at[pl.ds(sid * NPT, NPT)])
        pltpu.sync_copy(ut_hbm, ut_v)
        ut = ut_v[...]
        plsc.subcore_barrier()

        d_src, d_dst, d_ep, d_gx, d_gd = {}, {}, {}, {}, {}
        d_sc, d_cnt, d_r = {}, {}, {}

        def bb(k):
            return pl.multiple_of(wid * EW + k * C, C)

        def issue_loads(j):
            if j - 3 in d_sc:
                d_sc.pop(j - 3).wait()
            if j - 3 in d_cnt:
                d_cnt.pop(j - 3).wait()
            if j - 3 in d_r:
                d_r.pop(j - 3).wait()
            base = bb(j)
            s2, s3 = j % 2, j % 3
            d_src[j] = pltpu.async_copy(
                src_hbm.at[pl.ds(base, C)], src_v[s2], sem_src[s2])
            d_dst[j] = pltpu.async_copy(
                dst_hbm.at[pl.ds(base, C)], dst_v[s3], sem_dst[s3])
            d_ep[j] = pltpu.async_copy(
                ep_hbm.at[pl.ds(base // 8, C // 8)], ep_v[s2], sem_ep[s2])

        def issue_gathers(j):
            s2, s3 = j % 2, j % 3
            d_src.pop(j).wait()
            d_dst.pop(j).wait()
            d_gx[j] = pltpu.async_copy(
                xs_hbm.at[src_v[s2]], xs_v[s2], sem_gx[s2])
            d_gd[j] = pltpu.async_copy(
                xd_hbm.at[dst_v[s3]], xd_v[s2], sem_gd[s2])

        issue_loads(0)
        issue_gathers(0)
        for k in range(NCHUNK):
            s2, s3 = k % 2, k % 3
            if k + 1 < NCHUNK:
                issue_loads(k + 1)
            d_gx.pop(k).wait()
            d_gd.pop(k).wait()
            d_ep.pop(k).wait()
            if k + 1 < NCHUNK:
                issue_gathers(k + 1)

            xs_s, xd_s, ep_s, r_s = xs_v[s2], xd_v[s2], ep_v[s2], r_v[s3]
            rb_s = rb_v[s3]

            if with_count:
                @plsc.parallel_loop(0, C // 8, unroll=1)
                def body(row):
                    for j in range(8):
                        i = row * 8 + j
                        v = jnp.maximum(
                            xs_s[i] + xd_s[i]
                            + ep_s[row, pl.ds(L * j, L)] + ut, 0.0)
                        r_s[i] = v
                        rb_s[row, pl.ds(L * j, L)] = v
            else:
                @plsc.parallel_loop(0, C // 8, unroll=1)
                def body(row):
                    for j in range(8):
                        i = row * 8 + j
                        r_s[i] = jnp.maximum(
                            xs_s[i] + xd_s[i]
                            + ep_s[row, pl.ds(L * j, L)] + ut, 0.0)

            d_sc[k] = pltpu.async_copy(
                r_s, acc.at[dst_v[s3]], sem_sc[s3], add=True)
            if with_count:
                d_cnt[k] = pltpu.async_copy(
                    ones_v, cacc.at[dst_v[s3]], sem_cnt[s3], add=True)
                d_r[k] = pltpu.async_copy(
                    rb_s, r_hbm.at[pl.ds(bb(k) // 8, C // 8)], sem_r[s3])

        for j in sorted(d_sc):
            d_sc.pop(j).wait()
        for j in sorted(d_cnt):
            d_cnt.pop(j).wait()
        for j in sorted(d_r):
            d_r.pop(j).wait()

        plsc.subcore_barrier()
        pltpu.sync_copy(acc.at[pl.ds(sid * NPT, NPT)], z_v)
        pltpu.sync_copy(z_v, ssum_hbm.at[pl.ds(cid * NP + sid * NPT, NPT)])
        if with_count:
            pltpu.sync_copy(cacc.at[pl.ds(sid * NPT, NPT)], z_v)
            pltpu.sync_copy(z_v, cnt_hbm.at[pl.ds(cid * NP + sid * NPT, NPT)])

    return edge_sc


_edge_sc_full = _make_edge_sc(True)
_edge_sc_slim = _make_edge_sc(False)


_IDC = 250


def _make_ident_sc():
    mesh = plsc.VectorSubcoreMesh(core_axis_name="c", subcore_axis_name="s")
    rw = (E // 8) // NW
    out_type = [jax.ShapeDtypeStruct((E // 8, 8 * L), f32),
                jax.ShapeDtypeStruct((E // 8, 8 * L), f32)]
    scratch = ([pltpu.VMEM((_IDC, 8 * L), f32) for _ in range(2)]
               + [pltpu.SemaphoreType.DMA for _ in range(2)])

    @functools.partial(
        pl.kernel, mesh=mesh, out_type=out_type, scratch_types=scratch,
        compiler_params=pltpu.CompilerParams(use_tc_tiling_on_sc=False))
    def ident_sc(ea_hbm, eb_hbm, oa_hbm, ob_hbm, buf0, buf1, sem0, sem1):
        cid = lax.axis_index("c")
        sid = lax.axis_index("s")
        wid = sid * NC + cid
        for src_hbm, dst_hbm in ((ea_hbm, oa_hbm), (eb_hbm, ob_hbm)):
            for k in range(rw // _IDC):
                base = wid * rw + k * _IDC
                sl = pl.ds(base, _IDC)
                buf = (buf0, buf1)[k % 2]
                sem = (sem0, sem1)[k % 2]
                pltpu.async_copy(src_hbm.at[sl], buf, sem).wait()
                pltpu.async_copy(buf, dst_hbm.at[sl], sem).wait()

    return ident_sc


_ident_sc = _make_ident_sc()


def _proj_body(x_ref, wsd_ref, ucat_ref, w0u_ref, eb0_ref,
               xs_ref, xd_ref, ut_ref):
    pj = jnp.dot(x_ref[...], wsd_ref[...], preferred_element_type=f32)
    xs_ref[...] = pj[:, :L]
    xd_ref[...] = pj[:, L:]
    ut_ref[...] = (jnp.dot(ucat_ref[...], w0u_ref[...],
                           preferred_element_type=f32) + eb0_ref[...])


def _proj(x, wsd, ucat, w0u, eb0):
    return pl.pallas_call(
        _proj_body,
        out_shape=[jax.ShapeDtypeStruct((N, L), f32),
                   jax.ShapeDtypeStruct((N, L), f32),
                   jax.ShapeDtypeStruct((1, L), f32)],
    )(x, wsd, ucat, w0u, eb0)


_EP_CHUNK = 4000


def _ep_body(rows_ref, k_ref, pb_ref, o_ref):
    o_ref[...] = (jnp.dot(rows_ref[...], k_ref[...],
                          preferred_element_type=f32) + pb_ref[...])


_EPI_ROWS = 1000


def _ep_init_body(e_ref, w_ref, o_ref):
    for j in range(8):
        o_ref[:, L * j:L * (j + 1)] = jnp.dot(
            e_ref[:, j, :], w_ref[...], preferred_element_type=f32)


def _ep_init(e3, w0e):
    g = e3.shape[0] // _EPI_ROWS
    return pl.pallas_call(
        _ep_init_body,
        grid=(g,),
        in_specs=[pl.BlockSpec((_EPI_ROWS, 8, L), lambda i: (i, 0, 0)),
                  pl.BlockSpec((L, L), lambda i: (0, 0))],
        out_specs=pl.BlockSpec((_EPI_ROWS, 8 * L), lambda i: (i, 0)),
        out_shape=jax.ShapeDtypeStruct((e3.shape[0], 8 * L), f32),
    )(e3, w0e)


def _ep(rows_b, kmat, pb_tile):
    g = rows_b.shape[0] // _EP_CHUNK
    return pl.pallas_call(
        _ep_body,
        grid=(g,),
        in_specs=[pl.BlockSpec((_EP_CHUNK, 8 * L), lambda i: (i, 0)),
                  pl.BlockSpec((8 * L, 8 * L), lambda i: (0, 0)),
                  pl.BlockSpec((1, 8 * L), lambda i: (0, 0))],
        out_specs=pl.BlockSpec((_EP_CHUNK, 8 * L), lambda i: (i, 0)),
        out_shape=jax.ShapeDtypeStruct((rows_b.shape[0], 8 * L), f32),
    )(rows_b, kmat, pb_tile)


def _node_body(ssum_ref, cnt_ref, x_ref, ucat_ref, uoth_ref,
               eW1_ref, eb1_ref, Wx_ref, Wa_ref, Wu_ref, nb0_ref,
               nW1_ref, nb1_ref, Wgu_ref, Wgn_ref, gb0_ref, gW1_ref, gb1_ref,
               W0u_ref, eb0_ref, wsd_ref, oW0_ref, ob0_ref, oW1_ref, ob1_ref,
               xnew_ref, unew_ref, utnext_ref, xs_ref, xd_ref, out_ref):
    s = ssum_ref[:N, :] + ssum_ref[NP:NP + N, :]
    c = cnt_ref[:N, :1] + cnt_ref[NP:NP + N, :1]
    agg = ((jnp.dot(s, eW1_ref[...], preferred_element_type=f32)
            + c * eb1_ref[...]) / jnp.maximum(c, 1.0))
    ucat = ucat_ref[...]
    ut_n = jnp.dot(ucat, Wu_ref[...], preferred_element_type=f32) + nb0_ref[...]
    xh = jnp.maximum(
        jnp.dot(x_ref[...], Wx_ref[...], preferred_element_type=f32)
        + jnp.dot(agg, Wa_ref[...], preferred_element_type=f32) + ut_n, 0.0)
    x_new = jnp.dot(xh, nW1_ref[...], preferred_element_type=f32) + nb1_ref[...]
    xnew_ref[...] = x_new
    na = jnp.sum(x_new, axis=0, keepdims=True) * (1.0 / N)
    hg = jnp.maximum(
        jnp.dot(ucat, Wgu_ref[...], preferred_element_type=f32)
        + jnp.dot(na, Wgn_ref[...], preferred_element_type=f32)
        + gb0_ref[...], 0.0)
    u_new = jnp.dot(hg, gW1_ref[...], preferred_element_type=f32) + gb1_ref[...]
    unew_ref[...] = u_new
    uoth = uoth_ref[...]
    utnext_ref[...] = (
        jnp.dot(uoth, W0u_ref[:L, :], preferred_element_type=f32)
        + jnp.dot(u_new, W0u_ref[L:, :], preferred_element_type=f32)
        + eb0_ref[...])
    pj = jnp.dot(x_new, wsd_ref[...], preferred_element_type=f32)
    xs_ref[...] = pj[:, :L]
    xd_ref[...] = pj[:, L:]
    ho = jnp.maximum(
        jnp.dot(uoth, oW0_ref[:L, :], preferred_element_type=f32)
        + jnp.dot(u_new, oW0_ref[L:, :], preferred_element_type=f32)
        + ob0_ref[...], 0.0)
    out_ref[...] = (jnp.dot(ho, oW1_ref[...], preferred_element_type=f32)
                    + ob1_ref[...])


def _node(ssum, cnt, x, ucat, uoth, w):
    return pl.pallas_call(
        _node_body,
        out_shape=[jax.ShapeDtypeStruct((N, FX), f32),
                   jax.ShapeDtypeStruct((1, L), f32),
                   jax.ShapeDtypeStruct((1, L), f32),
                   jax.ShapeDtypeStruct((N, L), f32),
                   jax.ShapeDtypeStruct((N, L), f32),
                   jax.ShapeDtypeStruct((1, FOUT), f32)],
    )(ssum, cnt, x, ucat, uoth, *w)


def _node_last_body(ssum_ref, cnt_ref, x_ref, ucat_ref, uoth_ref,
                    eW1_ref, eb1_ref, Wx_ref, Wa_ref, Wu_ref, nb0_ref,
                    nW1_ref, nb1_ref, Wgu_ref, Wgn_ref, gb0_ref, gW1_ref,
                    gb1_ref, W0u_ref, eb0_ref, wsd_ref, oW0_ref, ob0_ref,
                    oW1_ref, ob1_ref, unew_ref, utnext_ref, out_ref):
    s = ssum_ref[:N, :] + ssum_ref[NP:NP + N, :]
    c = cnt_ref[:N, :1] + cnt_ref[NP:NP + N, :1]
    agg = ((jnp.dot(s, eW1_ref[...], preferred_element_type=f32)
            + c * eb1_ref[...]) / jnp.maximum(c, 1.0))
    ucat = ucat_ref[...]
    ut_n = jnp.dot(ucat, Wu_ref[...], preferred_element_type=f32) + nb0_ref[...]
    xh = jnp.maximum(
        jnp.dot(x_ref[...], Wx_ref[...], preferred_element_type=f32)
        + jnp.dot(agg, Wa_ref[...], preferred_element_type=f32) + ut_n, 0.0)
    x_new = jnp.dot(xh, nW1_ref[...], preferred_element_type=f32) + nb1_ref[...]
    na = jnp.sum(x_new, axis=0, keepdims=True) * (1.0 / N)
    hg = jnp.maximum(
        jnp.dot(ucat, Wgu_ref[...], preferred_element_type=f32)
        + jnp.dot(na, Wgn_ref[...], preferred_element_type=f32)
        + gb0_ref[...], 0.0)
    u_new = jnp.dot(hg, gW1_ref[...], preferred_element_type=f32) + gb1_ref[...]
    unew_ref[...] = u_new
    uoth = uoth_ref[...]
    utnext_ref[...] = (
        jnp.dot(uoth, W0u_ref[:L, :], preferred_element_type=f32)
        + jnp.dot(u_new, W0u_ref[L:, :], preferred_element_type=f32)
        + eb0_ref[...])
    ho = jnp.maximum(
        jnp.dot(uoth, oW0_ref[:L, :], preferred_element_type=f32)
        + jnp.dot(u_new, oW0_ref[L:, :], preferred_element_type=f32)
        + ob0_ref[...], 0.0)
    out_ref[...] = (jnp.dot(ho, oW1_ref[...], preferred_element_type=f32)
                    + ob1_ref[...])


def _node_last(ssum, cnt, x, ucat, uoth, w):
    return pl.pallas_call(
        _node_last_body,
        out_shape=[jax.ShapeDtypeStruct((1, L), f32),
                   jax.ShapeDtypeStruct((1, L), f32),
                   jax.ShapeDtypeStruct((1, FOUT), f32)],
    )(ssum, cnt, x, ucat, uoth, *w)


def kernel(x1, edge_index1, e1, u1, batch1, x2, edge_index2, e2, u2, batch2,
           edge_W0, edge_b0, edge_W1, edge_b1,
           node_W0, node_b0, node_W1, node_b1,
           glob_W0, glob_b0, glob_W1, glob_b1,
           out_W0, out_b0, out_W1, out_b1):
    src1, dst1 = edge_index1[0], edge_index1[1]
    src2, dst2 = edge_index2[0], edge_index2[1]

    wsd = jnp.concatenate([edge_W0[:FX], edge_W0[FX:2 * FX]], axis=1)
    w0e = edge_W0[2 * FX:2 * FX + FE]
    w0u = edge_W0[2 * FX + FE:]
    eb0 = edge_b0.reshape(1, L)
    eb1 = edge_b1.reshape(1, L)
    eye8 = jnp.eye(8, dtype=f32)
    k_init = jnp.kron(eye8, w0e)
    k_step = jnp.kron(eye8, edge_W1 @ w0e)
    pb_init = jnp.zeros((1, 8 * L), f32)
    pb_step = jnp.tile((edge_b1 @ w0e).reshape(1, L), (1, 8))
    wx = node_W0[:FX]
    wa = node_W0[FX:FX + FE]
    wu = node_W0[FX + FE:]
    nb0 = node_b0.reshape(1, L)
    nb1 = node_b1.reshape(1, FX)
    wgu = glob_W0[:2 * FU]
    wgn = glob_W0[2 * FU:]
    gb0 = glob_b0.reshape(1, L)
    gb1 = glob_b1.reshape(1, L)
    ob0 = out_b0.reshape(1, L)
    ob1 = out_b1.reshape(1, FOUT)
    nodew = (edge_W1, eb1, wx, wa, wu, nb0, node_W1, nb1,
             wgu, wgn, gb0, glob_W1, gb1, w0u, eb0, wsd,
             out_W0, ob0, out_W1, ob1)

    ucat11 = jnp.concatenate([u1, u2], axis=1)
    xs1, xd1, ut11 = _proj(x1, wsd, ucat11, w0u, eb0)
    xs2, xd2, _ = _proj(x2, wsd, ucat11, w0u, eb0)
    ep1 = _ep_init(e1.reshape(E // 8, 8, L), w0e)
    ep2 = _ep_init(e2.reshape(E // 8, 8, L), w0e)

    ssum1, r1, cnt1 = _edge_sc_full(src1, dst1, ep1, xs1, xd1,
                                    ut11.reshape(L))
    x1b, u1b, ut21, xs1b, xd1b, _ = _node(ssum1, cnt1, x1, ucat11, u2, nodew)

    ssum2, r2, cnt2 = _edge_sc_full(src2, dst2, ep2, xs2, xd2,
                                    ut21.reshape(L))
    ucat21 = jnp.concatenate([u2, u1b], axis=1)
    x2b, u2b, ut12, xs2b, xd2b, out1 = _node(ssum2, cnt2, x2, ucat21, u1b,
                                             nodew)

    ep1b = _ep(r1, k_step, pb_step)
    (ssum1b,) = _edge_sc_slim(src1, dst1, ep1b, xs1b, xd1b, ut12.reshape(L))
    ucat12 = jnp.concatenate([u1b, u2b], axis=1)
    u1c, ut22, _ = _node_last(ssum1b, cnt1, x1b, ucat12, u2b, nodew)

    ep2b = _ep(r2, k_step, pb_step)
    (ssum2b,) = _edge_sc_slim(src2, dst2, ep2b, xs2b, xd2b, ut22.reshape(L))
    ucat22 = jnp.concatenate([u2b, u1c], axis=1)
    _, _, out2 = _node_last(ssum2b, cnt2, x2b, ucat22, u1c, nodew)

    return jnp.stack([out1, out2])


# speedup vs baseline: 1.1308x; 1.0088x over previous
import functools

import jax
import jax.numpy as jnp
from jax import lax
from jax.experimental import pallas as pl
from jax.experimental.pallas import tpu as pltpu
from jax.experimental.pallas import tpu_sc as plsc

FE, FX, FU, FOUT = 16, 128, 16, 2
N, E = 10000, 320000
L = 16
NC, NS = 2, 16
NW = NC * NS
EW = E // NW
C = 400
NCHUNK = EW // C
NP = 10240
NPT = NP // NS

f32 = jnp.float32


def _make_edge_sc(with_count):
    mesh = plsc.VectorSubcoreMesh(core_axis_name="c", subcore_axis_name="s")

    out_type = [jax.ShapeDtypeStruct((NC * NP, L), f32)]
    if with_count:
        out_type.append(jax.ShapeDtypeStruct((E // 8, 8 * L), f32))

    scratch = (
        [pltpu.VMEM((C,), jnp.int32) for _ in range(2)]
        + [pltpu.VMEM((C,), jnp.int32) for _ in range(3)]
        + [pltpu.VMEM((C // 8, 8 * L), f32) for _ in range(2)]
        + [pltpu.VMEM((C, L), f32) for _ in range(2)]
        + [pltpu.VMEM((C, L), f32) for _ in range(2)]
        + [pltpu.VMEM((C, L), f32) for _ in range(3)]
        + [pltpu.VMEM((C // 8, 8 * L), f32) for _ in range(3)]
        + [
            pltpu.VMEM((L,), f32),
            pltpu.VMEM((NPT, L), f32),
            pltpu.VMEM_SHARED((NP, L), f32),
        ]
    )
    n_sem = 2 + 3 + 2 + 2 + 2 + 3
    if with_count:
        n_sem += 3
    scratch += [pltpu.SemaphoreType.DMA for _ in range(n_sem)]

    @functools.partial(
        pl.kernel, mesh=mesh, out_type=out_type, scratch_types=scratch,
        compiler_params=pltpu.CompilerParams(use_tc_tiling_on_sc=False))
    def edge_sc(src_hbm, dst_hbm, ep_hbm, xs_hbm, xd_hbm, ut_hbm, *rest):
        if with_count:
            ssum_hbm, r_hbm = rest[:2]
            rest = rest[2:]
        else:
            ssum_hbm = rest[0]
            rest = rest[1:]
        src_v = rest[0:2]
        dst_v = rest[2:5]
        ep_v = rest[5:7]
        xs_v = rest[7:9]
        xd_v = rest[9:11]
        r_v = rest[11:14]
        rb_v = rest[14:17]
        ut_v, z_v, acc = rest[17:20]
        rest = rest[20:]
        sem_src = rest[0:2]
        sem_dst = rest[2:5]
        sem_ep = rest[5:7]
        sem_gx = rest[7:9]
        sem_gd = rest[9:11]
        sem_sc = rest[11:14]
        if with_count:
            sem_r = rest[14:17]

        cid = lax.axis_index("c")
        sid = lax.axis_index("s")
        wid = sid * NC + cid

        def zf(i, carry):
            z_v[i] = jnp.zeros((L,), f32)
            return carry
        lax.fori_loop(0, NPT, zf, 0)
        pltpu.sync_copy(z_v, acc.at[pl.ds(sid * NPT, NPT)])
        pltpu.sync_copy(ut_hbm, ut_v)
        ut = ut_v[...]
        plsc.subcore_barrier()

        d_src, d_dst, d_ep, d_gx, d_gd = {}, {}, {}, {}, {}
        d_sc, d_cnt, d_r = {}, {}, {}

        def bb(k):
            return pl.multiple_of(wid * EW + k * C, C)

        def issue_loads(j):
            if j - 3 in d_sc:
                d_sc.pop(j - 3).wait()
            if j - 3 in d_cnt:
                d_cnt.pop(j - 3).wait()
            if j - 3 in d_r:
                d_r.pop(j - 3).wait()
            base = bb(j)
            s2, s3 = j % 2, j % 3
            d_src[j] = pltpu.async_copy(
                src_hbm.at[pl.ds(base, C)], src_v[s2], sem_src[s2])
            d_dst[j] = pltpu.async_copy(
                dst_hbm.at[pl.ds(base, C)], dst_v[s3], sem_dst[s3])
            d_ep[j] = pltpu.async_copy(
                ep_hbm.at[pl.ds(base // 8, C // 8)], ep_v[s2], sem_ep[s2])

        def issue_gathers(j):
            s2, s3 = j % 2, j % 3
            d_src.pop(j).wait()
            d_dst.pop(j).wait()
            d_gx[j] = pltpu.async_copy(
                xs_hbm.at[src_v[s2]], xs_v[s2], sem_gx[s2])
            d_gd[j] = pltpu.async_copy(
                xd_hbm.at[dst_v[s3]], xd_v[s2], sem_gd[s2])

        issue_loads(0)
        issue_gathers(0)
        for k in range(NCHUNK):
            s2, s3 = k % 2, k % 3
            if k + 1 < NCHUNK:
                issue_loads(k + 1)
            d_gx.pop(k).wait()
            d_gd.pop(k).wait()
            d_ep.pop(k).wait()
            if k + 1 < NCHUNK:
                issue_gathers(k + 1)

            xs_s, xd_s, ep_s, r_s = xs_v[s2], xd_v[s2], ep_v[s2], r_v[s3]
            rb_s = rb_v[s3]

            if with_count:
                @plsc.parallel_loop(0, C // 8, unroll=1)
                def body(row):
                    for j in range(8):
                        i = row * 8 + j
                        v = jnp.maximum(
                            xs_s[i] + xd_s[i]
                            + ep_s[row, pl.ds(L * j, L)] + ut, 0.0)
                        r_s[i] = v
                        rb_s[row, pl.ds(L * j, L)] = v
            else:
                @plsc.parallel_loop(0, C // 8, unroll=1)
                def body(row):
                    for j in range(8):
                        i = row * 8 + j
                        r_s[i] = jnp.maximum(
                            xs_s[i] + xd_s[i]
                            + ep_s[row, pl.ds(L * j, L)] + ut, 0.0)

            d_sc[k] = pltpu.async_copy(
                r_s, acc.at[dst_v[s3]], sem_sc[s3], add=True)
            if with_count:
                d_r[k] = pltpu.async_copy(
                    rb_s, r_hbm.at[pl.ds(bb(k) // 8, C // 8)], sem_r[s3])

        for j in sorted(d_sc):
            d_sc.pop(j).wait()
        for j in sorted(d_cnt):
            d_cnt.pop(j).wait()
        for j in sorted(d_r):
            d_r.pop(j).wait()

        plsc.subcore_barrier()
        pltpu.sync_copy(acc.at[pl.ds(sid * NPT, NPT)], z_v)
        pltpu.sync_copy(z_v, ssum_hbm.at[pl.ds(cid * NP + sid * NPT, NPT)])

    return edge_sc


_edge_sc_full = _make_edge_sc(True)
_edge_sc_slim = _make_edge_sc(False)


def _make_count_sc():
    mesh = plsc.VectorSubcoreMesh(core_axis_name="c", subcore_axis_name="s")
    CC = 2000
    out_type = [jax.ShapeDtypeStruct((NC * NP, L), f32),
                jax.ShapeDtypeStruct((NC * NP, L), f32)]
    scratch = (
        [pltpu.VMEM((CC,), jnp.int32) for _ in range(2)]
        + [pltpu.VMEM((CC, L), f32),
           pltpu.VMEM((NPT, L), f32),
           pltpu.VMEM_SHARED((NP, L), f32),
           pltpu.VMEM_SHARED((NP, L), f32)]
        + [pltpu.SemaphoreType.DMA for _ in range(4)]
    )

    @functools.partial(
        pl.kernel, mesh=mesh, out_type=out_type, scratch_types=scratch,
        compiler_params=pltpu.CompilerParams(use_tc_tiling_on_sc=False))
    def count_sc(dst1_hbm, dst2_hbm, c1_hbm, c2_hbm,
                 d0, d1, ones_v, z_v, acc1, acc2, s0, s1, s2, s3):
        cid = lax.axis_index("c")
        sid = lax.axis_index("s")
        wid = sid * NC + cid

        def zf(i, carry):
            z_v[i] = jnp.zeros((L,), f32)
            return carry
        lax.fori_loop(0, NPT, zf, 0)

        def of(i, carry):
            ones_v[i] = jnp.full((L,), 1.0, f32)
            return carry
        lax.fori_loop(0, CC, of, 0)
        pltpu.sync_copy(z_v, acc1.at[pl.ds(sid * NPT, NPT)])
        pltpu.sync_copy(z_v, acc2.at[pl.ds(sid * NPT, NPT)])
        plsc.subcore_barrier()

        nchunk = EW // CC
        for dst_hbm, accx, sld, ssc in ((dst1_hbm, acc1, s0, s1),
                                        (dst2_hbm, acc2, s2, s3)):
            prev = None
            for k in range(nchunk):
                base = pl.multiple_of(wid * EW + k * CC, CC)
                dv = (d0, d1)[k % 2]
                pltpu.async_copy(dst_hbm.at[pl.ds(base, CC)], dv, sld).wait()
                if prev is not None:
                    prev.wait()
                prev = pltpu.async_copy(ones_v, accx.at[dv], ssc, add=True)
            prev.wait()

        plsc.subcore_barrier()
        pltpu.sync_copy(acc1.at[pl.ds(sid * NPT, NPT)], z_v)
        pltpu.sync_copy(z_v, c1_hbm.at[pl.ds(cid * NP + sid * NPT, NPT)])
        pltpu.sync_copy(acc2.at[pl.ds(sid * NPT, NPT)], z_v)
        pltpu.sync_copy(z_v, c2_hbm.at[pl.ds(cid * NP + sid * NPT, NPT)])

    return count_sc


_count_sc = _make_count_sc()


def _proj_body(x_ref, wsd_ref, ucat_ref, w0u_ref, eb0_ref,
               xs_ref, xd_ref, ut_ref):
    pj = jnp.dot(x_ref[...], wsd_ref[...], preferred_element_type=f32)
    xs_ref[...] = pj[:, :L]
    xd_ref[...] = pj[:, L:]
    ut_ref[...] = (jnp.dot(ucat_ref[...], w0u_ref[...],
                           preferred_element_type=f32) + eb0_ref[...])


def _proj(x, wsd, ucat, w0u, eb0):
    return pl.pallas_call(
        _proj_body,
        out_shape=[jax.ShapeDtypeStruct((N, L), f32),
                   jax.ShapeDtypeStruct((N, L), f32),
                   jax.ShapeDtypeStruct((1, L), f32)],
    )(x, wsd, ucat, w0u, eb0)


_EP_CHUNK = 4000


def _ep_body(rows_ref, k_ref, pb_ref, o_ref):
    o_ref[...] = (jnp.dot(rows_ref[...], k_ref[...],
                          preferred_element_type=f32) + pb_ref[...])


_EPI_ROWS = 1000


def _ep_init_body(e_ref, w_ref, o_ref):
    for j in range(8):
        o_ref[:, L * j:L * (j + 1)] = jnp.dot(
            e_ref[:, j, :], w_ref[...], preferred_element_type=f32)


def _ep_init(e3, w0e):
    g = e3.shape[0] // _EPI_ROWS
    return pl.pallas_call(
        _ep_init_body,
        grid=(g,),
        in_specs=[pl.BlockSpec((_EPI_ROWS, 8, L), lambda i: (i, 0, 0)),
                  pl.BlockSpec((L, L), lambda i: (0, 0))],
        out_specs=pl.BlockSpec((_EPI_ROWS, 8 * L), lambda i: (i, 0)),
        out_shape=jax.ShapeDtypeStruct((e3.shape[0], 8 * L), f32),
    )(e3, w0e)


def _ep(rows_b, kmat, pb_tile):
    g = rows_b.shape[0] // _EP_CHUNK
    return pl.pallas_call(
        _ep_body,
        grid=(g,),
        in_specs=[pl.BlockSpec((_EP_CHUNK, 8 * L), lambda i: (i, 0)),
                  pl.BlockSpec((8 * L, 8 * L), lambda i: (0, 0)),
                  pl.BlockSpec((1, 8 * L), lambda i: (0, 0))],
        out_specs=pl.BlockSpec((_EP_CHUNK, 8 * L), lambda i: (i, 0)),
        out_shape=jax.ShapeDtypeStruct((rows_b.shape[0], 8 * L), f32),
    )(rows_b, kmat, pb_tile)


def _node_body(ssum_ref, cnt_ref, x_ref, ucat_ref, uoth_ref,
               eW1_ref, eb1_ref, Wx_ref, Wa_ref, Wu_ref, nb0_ref,
               nW1_ref, nb1_ref, Wgu_ref, Wgn_ref, gb0_ref, gW1_ref, gb1_ref,
               W0u_ref, eb0_ref, wsd_ref, oW0_ref, ob0_ref, oW1_ref, ob1_ref,
               xnew_ref, unew_ref, utnext_ref, xs_ref, xd_ref, out_ref):
    s = ssum_ref[:N, :] + ssum_ref[NP:NP + N, :]
    c = cnt_ref[:N, :1] + cnt_ref[NP:NP + N, :1]
    agg = ((jnp.dot(s, eW1_ref[...], preferred_element_type=f32)
            + c * eb1_ref[...]) / jnp.maximum(c, 1.0))
    ucat = ucat_ref[...]
    ut_n = jnp.dot(ucat, Wu_ref[...], preferred_element_type=f32) + nb0_ref[...]
    xh = jnp.maximum(
        jnp.dot(x_ref[...], Wx_ref[...], preferred_element_type=f32)
        + jnp.dot(agg, Wa_ref[...], preferred_element_type=f32) + ut_n, 0.0)
    x_new = jnp.dot(xh, nW1_ref[...], preferred_element_type=f32) + nb1_ref[...]
    xnew_ref[...] = x_new
    na = jnp.sum(x_new, axis=0, keepdims=True) * (1.0 / N)
    hg = jnp.maximum(
        jnp.dot(ucat, Wgu_ref[...], preferred_element_type=f32)
        + jnp.dot(na, Wgn_ref[...], preferred_element_type=f32)
        + gb0_ref[...], 0.0)
    u_new = jnp.dot(hg, gW1_ref[...], preferred_element_type=f32) + gb1_ref[...]
    unew_ref[...] = u_new
    uoth = uoth_ref[...]
    utnext_ref[...] = (
        jnp.dot(uoth, W0u_ref[:L, :], preferred_element_type=f32)
        + jnp.dot(u_new, W0u_ref[L:, :], preferred_element_type=f32)
        + eb0_ref[...])
    pj = jnp.dot(x_new, wsd_ref[...], preferred_element_type=f32)
    xs_ref[...] = pj[:, :L]
    xd_ref[...] = pj[:, L:]
    ho = jnp.maximum(
        jnp.dot(uoth, oW0_ref[:L, :], preferred_element_type=f32)
        + jnp.dot(u_new, oW0_ref[L:, :], preferred_element_type=f32)
        + ob0_ref[...], 0.0)
    out_ref[...] = (jnp.dot(ho, oW1_ref[...], preferred_element_type=f32)
                    + ob1_ref[...])


def _node(ssum, cnt, x, ucat, uoth, w):
    return pl.pallas_call(
        _node_body,
        out_shape=[jax.ShapeDtypeStruct((N, FX), f32),
                   jax.ShapeDtypeStruct((1, L), f32),
                   jax.ShapeDtypeStruct((1, L), f32),
                   jax.ShapeDtypeStruct((N, L), f32),
                   jax.ShapeDtypeStruct((N, L), f32),
                   jax.ShapeDtypeStruct((1, FOUT), f32)],
    )(ssum, cnt, x, ucat, uoth, *w)


def _node_last_body(ssum_ref, cnt_ref, x_ref, ucat_ref, uoth_ref,
                    eW1_ref, eb1_ref, Wx_ref, Wa_ref, Wu_ref, nb0_ref,
                    nW1_ref, nb1_ref, Wgu_ref, Wgn_ref, gb0_ref, gW1_ref,
                    gb1_ref, W0u_ref, eb0_ref, wsd_ref, oW0_ref, ob0_ref,
                    oW1_ref, ob1_ref, unew_ref, utnext_ref, out_ref):
    s = ssum_ref[:N, :] + ssum_ref[NP:NP + N, :]
    c = cnt_ref[:N, :1] + cnt_ref[NP:NP + N, :1]
    agg = ((jnp.dot(s, eW1_ref[...], preferred_element_type=f32)
            + c * eb1_ref[...]) / jnp.maximum(c, 1.0))
    ucat = ucat_ref[...]
    ut_n = jnp.dot(ucat, Wu_ref[...], preferred_element_type=f32) + nb0_ref[...]
    xh = jnp.maximum(
        jnp.dot(x_ref[...], Wx_ref[...], preferred_element_type=f32)
        + jnp.dot(agg, Wa_ref[...], preferred_element_type=f32) + ut_n, 0.0)
    x_new = jnp.dot(xh, nW1_ref[...], preferred_element_type=f32) + nb1_ref[...]
    na = jnp.sum(x_new, axis=0, keepdims=True) * (1.0 / N)
    hg = jnp.maximum(
        jnp.dot(ucat, Wgu_ref[...], preferred_element_type=f32)
        + jnp.dot(na, Wgn_ref[...], preferred_element_type=f32)
        + gb0_ref[...], 0.0)
    u_new = jnp.dot(hg, gW1_ref[...], preferred_element_type=f32) + gb1_ref[...]
    unew_ref[...] = u_new
    uoth = uoth_ref[...]
    utnext_ref[...] = (
        jnp.dot(uoth, W0u_ref[:L, :], preferred_element_type=f32)
        + jnp.dot(u_new, W0u_ref[L:, :], preferred_element_type=f32)
        + eb0_ref[...])
    ho = jnp.maximum(
        jnp.dot(uoth, oW0_ref[:L, :], preferred_element_type=f32)
        + jnp.dot(u_new, oW0_ref[L:, :], preferred_element_type=f32)
        + ob0_ref[...], 0.0)
    out_ref[...] = (jnp.dot(ho, oW1_ref[...], preferred_element_type=f32)
                    + ob1_ref[...])


def _node_last(ssum, cnt, x, ucat, uoth, w):
    return pl.pallas_call(
        _node_last_body,
        out_shape=[jax.ShapeDtypeStruct((1, L), f32),
                   jax.ShapeDtypeStruct((1, L), f32),
                   jax.ShapeDtypeStruct((1, FOUT), f32)],
    )(ssum, cnt, x, ucat, uoth, *w)


def kernel(x1, edge_index1, e1, u1, batch1, x2, edge_index2, e2, u2, batch2,
           edge_W0, edge_b0, edge_W1, edge_b1,
           node_W0, node_b0, node_W1, node_b1,
           glob_W0, glob_b0, glob_W1, glob_b1,
           out_W0, out_b0, out_W1, out_b1):
    src1, dst1 = edge_index1[0], edge_index1[1]
    src2, dst2 = edge_index2[0], edge_index2[1]

    wsd = jnp.concatenate([edge_W0[:FX], edge_W0[FX:2 * FX]], axis=1)
    w0e = edge_W0[2 * FX:2 * FX + FE]
    w0u = edge_W0[2 * FX + FE:]
    eb0 = edge_b0.reshape(1, L)
    eb1 = edge_b1.reshape(1, L)
    eye8 = jnp.eye(8, dtype=f32)
    k_init = jnp.kron(eye8, w0e)
    k_step = jnp.kron(eye8, edge_W1 @ w0e)
    pb_init = jnp.zeros((1, 8 * L), f32)
    pb_step = jnp.tile((edge_b1 @ w0e).reshape(1, L), (1, 8))
    wx = node_W0[:FX]
    wa = node_W0[FX:FX + FE]
    wu = node_W0[FX + FE:]
    nb0 = node_b0.reshape(1, L)
    nb1 = node_b1.reshape(1, FX)
    wgu = glob_W0[:2 * FU]
    wgn = glob_W0[2 * FU:]
    gb0 = glob_b0.reshape(1, L)
    gb1 = glob_b1.reshape(1, L)
    ob0 = out_b0.reshape(1, L)
    ob1 = out_b1.reshape(1, FOUT)
    nodew = (edge_W1, eb1, wx, wa, wu, nb0, node_W1, nb1,
             wgu, wgn, gb0, glob_W1, gb1, w0u, eb0, wsd,
             out_W0, ob0, out_W1, ob1)

    ucat11 = jnp.concatenate([u1, u2], axis=1)
    cnt1, cnt2 = _count_sc(dst1, dst2)
    xs1, xd1, ut11 = _proj(x1, wsd, ucat11, w0u, eb0)
    xs2, xd2, _ = _proj(x2, wsd, ucat11, w0u, eb0)
    ep1 = _ep_init(e1.reshape(E // 8, 8, L), w0e)
    ep2 = _ep_init(e2.reshape(E // 8, 8, L), w0e)

    ssum1, r1 = _edge_sc_full(src1, dst1, ep1, xs1, xd1, ut11.reshape(L))
    x1b, u1b, ut21, xs1b, xd1b, _ = _node(ssum1, cnt1, x1, ucat11, u2, nodew)

    ssum2, r2 = _edge_sc_full(src2, dst2, ep2, xs2, xd2, ut21.reshape(L))
    ucat21 = jnp.concatenate([u2, u1b], axis=1)
    x2b, u2b, ut12, xs2b, xd2b, out1 = _node(ssum2, cnt2, x2, ucat21, u1b,
                                             nodew)

    ep1b = _ep(r1, k_step, pb_step)
    (ssum1b,) = _edge_sc_slim(src1, dst1, ep1b, xs1b, xd1b, ut12.reshape(L))
    ucat12 = jnp.concatenate([u1b, u2b], axis=1)
    u1c, ut22, _ = _node_last(ssum1b, cnt1, x1b, ucat12, u2b, nodew)

    ep2b = _ep(r2, k_step, pb_step)
    (ssum2b,) = _edge_sc_slim(src2, dst2, ep2b, xs2b, xd2b, ut22.reshape(L))
    ucat22 = jnp.concatenate([u2b, u1c], axis=1)
    _, _, out2 = _node_last(ssum2b, cnt2, x2b, ucat22, u1c, nodew)

    return jnp.stack([out1, out2])
